# baseline (speedup 1.0000x reference)
import functools

import jax
import jax.numpy as jnp
from jax import lax
from jax.experimental import pallas as pl
from jax.experimental.pallas import tpu as pltpu
from jax.experimental.pallas import tpu_sc as plsc

NUM_NODES = 10000
NUM_EDGES = 320000
D = 128
HALF = NUM_NODES // 2
ACC_ROWS = NUM_NODES + 8

NC = 2
NS = 16

CHUNK = 128
NUM_CHUNKS = NUM_EDGES // CHUNK
NBUF = 3
ROWS_PT = 320
ROWS_PT_LAST = HALF - ROWS_PT * (NS - 1)


def _split_and_zeros(ids2d):
    def body(ids_ref, o_ref, z_ref):
        s = jnp.sum((ids_ref[...] < HALF).astype(jnp.int32))
        o_ref[...] = jnp.full((1, 16), s, jnp.int32)
        z_ref[...] = jnp.zeros((ROWS_PT, D), jnp.float32)

    out, zeros2d = pl.pallas_call(
        body,
        out_shape=[
            jax.ShapeDtypeStruct((1, 16), jnp.int32),
            jax.ShapeDtypeStruct((ROWS_PT, D), jnp.float32),
        ],
    )(ids2d)
    return out.reshape(16), zeros2d


def _sc_segment_sum(E2d, ids3d, zeros2d, split16):
    mesh = plsc.VectorSubcoreMesh(core_axis_name="c", subcore_axis_name="s")

    @functools.partial(
        pl.kernel,
        out_type=jax.ShapeDtypeStruct((NUM_NODES, D), jnp.float32),
        mesh=mesh,
        scratch_types=[
            pltpu.VMEM((NBUF, CHUNK, D), jnp.float32),
            pltpu.VMEM((NBUF, 1, CHUNK), jnp.int32),
            pltpu.VMEM_SHARED((ACC_ROWS, D), jnp.float32),
            pltpu.VMEM((16,), jnp.int32),
            pltpu.SemaphoreType.DMA((NBUF,)),
            pltpu.SemaphoreType.DMA((NBUF,)),
            pltpu.SemaphoreType.DMA((NBUF,)),
        ],
    )
    def k(e_hbm, ids_hbm, zeros_hbm, split_hbm, out_hbm,
          rows_v, idx_v, acc_s, split_m, sem_r, sem_i, sem_s):
        cid = lax.axis_index("c")
        sid = lax.axis_index("s")

        pltpu.sync_copy(split_hbm, split_m)
        split = split_m[...][0]
        lo_c = jnp.where(cid == 0, 0, split // CHUNK)
        hi_c = jnp.where(cid == 0, (split + CHUNK - 1) // CHUNK, NUM_CHUNKS)
        base = cid * HALF

        @pl.when(sid < NS - 1)
        def _():
            pltpu.sync_copy(
                zeros_hbm,
                acc_s.at[pl.ds(base + sid * ROWS_PT, ROWS_PT)],
            )

        @pl.when(sid == NS - 1)
        def _():
            pltpu.sync_copy(
                zeros_hbm.at[pl.ds(0, ROWS_PT_LAST)],
                acc_s.at[pl.ds(base + (NS - 1) * ROWS_PT, ROWS_PT_LAST)],
            )

        plsc.subcore_barrier()

        def start(it, b):
            j = lo_c + it * NS + sid

            @pl.when(j < hi_c)
            def _():
                pltpu.async_copy(ids_hbm.at[j], idx_v.at[b], sem_i.at[b])
                pltpu.async_copy(
                    e_hbm.at[pl.ds(j * CHUNK, CHUNK)], rows_v.at[b], sem_r.at[b]
                )

        def finish(it, b):
            j = lo_c + it * NS + sid

            @pl.when(j < hi_c)
            def _():
                pltpu.make_async_copy(ids_hbm.at[j], idx_v.at[b], sem_i.at[b]).wait()
                pltpu.make_async_copy(
                    e_hbm.at[pl.ds(j * CHUNK, CHUNK)], rows_v.at[b], sem_r.at[b]
                ).wait()
                pltpu.async_copy(
                    rows_v.at[b], acc_s.at[idx_v.at[b, 0]], sem_s.at[b], add=True
                )

        def drain(it, b):
            j = lo_c + it * NS + sid

            @pl.when((it >= 0) & (j < hi_c))
            def _():
                pltpu.make_async_copy(
                    rows_v.at[b], acc_s.at[idx_v.at[b, 0]], sem_s.at[b]
                ).wait()

        start(0, 0)
        start(1, 1)

        my_n = jnp.maximum(hi_c - lo_c - sid + NS - 1, 0) // NS

        def body(kk, _):
            for b in range(NBUF):
                it = kk * NBUF + b
                finish(it, b)
                drain(it - 1, (b - 1) % NBUF)
                start(it + 2, (b + 2) % NBUF)
            return ()

        lax.fori_loop(0, (my_n + NBUF - 1) // NBUF + 1, body, ())
        plsc.subcore_barrier()

        @pl.when(sid < NS - 1)
        def _():
            pltpu.sync_copy(
                acc_s.at[pl.ds(base + sid * ROWS_PT, ROWS_PT)],
                out_hbm.at[pl.ds(base + sid * ROWS_PT, ROWS_PT)],
            )

        @pl.when(sid == NS - 1)
        def _():
            pltpu.sync_copy(
                acc_s.at[pl.ds(base + (NS - 1) * ROWS_PT, ROWS_PT_LAST)],
                out_hbm.at[pl.ds(base + (NS - 1) * ROWS_PT, ROWS_PT_LAST)],
            )

    return k(E2d, ids3d, zeros2d, split16)


@jax.jit
def kernel(V_set, E_set, node_ids):
    E2d = E_set[0]
    ids3d = node_ids[0].reshape(NUM_CHUNKS, 1, CHUNK)
    split16, zeros2d = _split_and_zeros(node_ids[0].reshape(NUM_EDGES // D, D))
    out = _sc_segment_sum(E2d, ids3d, zeros2d, split16)
    return out[jnp.newaxis]


# speedup vs baseline: 1.0773x; 1.0773x over previous
import functools

import jax
import jax.numpy as jnp
from jax import lax
from jax.experimental import pallas as pl
from jax.experimental.pallas import tpu as pltpu
from jax.experimental.pallas import tpu_sc as plsc

NUM_NODES = 10000
NUM_EDGES = 320000
D = 128
HALF = NUM_NODES // 2
ACC_ROWS = NUM_NODES + 8

NC = 2
NS = 16

CHUNK = 128
NUM_CHUNKS = NUM_EDGES // CHUNK
NBUF = 3
ROWS_PT = 320
ROWS_PT_LAST = HALF - ROWS_PT * (NS - 1)


def _split_and_zeros(ids2d):
    def body(ids_ref, o_ref, z_ref):
        s = jnp.sum((ids_ref[...] < HALF).astype(jnp.int32))
        o_ref[...] = jnp.full((1, 16), s, jnp.int32)
        z_ref[...] = jnp.zeros((ROWS_PT, D), jnp.float32)

    out, zeros2d = pl.pallas_call(
        body,
        out_shape=[
            jax.ShapeDtypeStruct((1, 16), jnp.int32),
            jax.ShapeDtypeStruct((ROWS_PT, D), jnp.float32),
        ],
    )(ids2d)
    return out.reshape(16), zeros2d


def _sc_segment_sum(E2d, ids3d, zeros2d, split16):
    mesh = plsc.VectorSubcoreMesh(core_axis_name="c", subcore_axis_name="s")

    @functools.partial(
        pl.kernel,
        out_type=jax.ShapeDtypeStruct((NUM_NODES, D), jnp.float32),
        mesh=mesh,
        scratch_types=[
            pltpu.VMEM((NBUF, CHUNK, D), jnp.float32),
            pltpu.VMEM((NBUF, 1, CHUNK), jnp.int32),
            pltpu.VMEM_SHARED((ACC_ROWS, D), jnp.float32),
            pltpu.VMEM((16,), jnp.int32),
            pltpu.SemaphoreType.DMA((NBUF,)),
            pltpu.SemaphoreType.DMA((NBUF,)),
        ],
    )
    def k(e_hbm, ids_hbm, zeros_hbm, split_hbm, out_hbm,
          rows_v, idx_v, acc_s, split_m, sem_r, sem_i):
        cid = lax.axis_index("c")
        sid = lax.axis_index("s")

        pltpu.sync_copy(split_hbm, split_m)
        split = split_m[...][0]
        lo_c = jnp.where(cid == 0, 0, split // CHUNK)
        hi_c = jnp.where(cid == 0, (split + CHUNK - 1) // CHUNK, NUM_CHUNKS)
        base = cid * HALF

        @pl.when(sid < NS - 1)
        def _():
            pltpu.sync_copy(
                zeros_hbm,
                acc_s.at[pl.ds(base + sid * ROWS_PT, ROWS_PT)],
            )

        @pl.when(sid == NS - 1)
        def _():
            pltpu.sync_copy(
                zeros_hbm.at[pl.ds(0, ROWS_PT_LAST)],
                acc_s.at[pl.ds(base + (NS - 1) * ROWS_PT, ROWS_PT_LAST)],
            )

        plsc.subcore_barrier()

        def start(it, b):
            j = lo_c + it * NS + sid

            @pl.when(j < hi_c)
            def _():
                pltpu.async_copy(ids_hbm.at[j], idx_v.at[b], sem_i.at[b])
                pltpu.async_copy(
                    e_hbm.at[pl.ds(j * CHUNK, CHUNK)], rows_v.at[b], sem_r.at[b]
                )

        def finish(it, b):
            j = lo_c + it * NS + sid

            @pl.when(j < hi_c)
            def _():
                pltpu.make_async_copy(ids_hbm.at[j], idx_v.at[b], sem_i.at[b]).wait()
                pltpu.make_async_copy(
                    e_hbm.at[pl.ds(j * CHUNK, CHUNK)], rows_v.at[b], sem_r.at[b]
                ).wait()
                pltpu.sync_copy(rows_v.at[b], acc_s.at[idx_v.at[b, 0]], add=True)

        start(0, 0)
        start(1, 1)

        my_n = jnp.maximum(hi_c - lo_c - sid + NS - 1, 0) // NS

        def body(kk, _):
            for b in range(NBUF):
                it = kk * NBUF + b
                start(it + 2, (b + 2) % NBUF)
                finish(it, b)
            return ()

        lax.fori_loop(0, (my_n + NBUF - 1) // NBUF + 1, body, ())
        plsc.subcore_barrier()

        @pl.when(sid < NS - 1)
        def _():
            pltpu.sync_copy(
                acc_s.at[pl.ds(base + sid * ROWS_PT, ROWS_PT)],
                out_hbm.at[pl.ds(base + sid * ROWS_PT, ROWS_PT)],
            )

        @pl.when(sid == NS - 1)
        def _():
            pltpu.sync_copy(
                acc_s.at[pl.ds(base + (NS - 1) * ROWS_PT, ROWS_PT_LAST)],
                out_hbm.at[pl.ds(base + (NS - 1) * ROWS_PT, ROWS_PT_LAST)],
            )

    return k(E2d, ids3d, zeros2d, split16)


@jax.jit
def kernel(V_set, E_set, node_ids):
    E2d = E_set[0]
    ids3d = node_ids[0].reshape(NUM_CHUNKS, 1, CHUNK)
    split16, zeros2d = _split_and_zeros(node_ids[0].reshape(NUM_EDGES // D, D))
    out = _sc_segment_sum(E2d, ids3d, zeros2d, split16)
    return out[jnp.newaxis]


# speedup vs baseline: 1.0952x; 1.0167x over previous
import functools

import jax
import jax.numpy as jnp
from jax import lax
from jax.experimental import pallas as pl
from jax.experimental.pallas import tpu as pltpu
from jax.experimental.pallas import tpu_sc as plsc

NUM_NODES = 10000
NUM_EDGES = 320000
D = 128
HALF = NUM_NODES // 2
ACC_ROWS = NUM_NODES + 8

NC = 2
NS = 16

CHUNK = 128
NUM_CHUNKS = NUM_EDGES // CHUNK
NBUF = 3
ROWS_PT = 320
ROWS_PT_LAST = HALF - ROWS_PT * (NS - 1)


def _split_and_zeros(ids2d):
    def body(ids_ref, o_ref, z_ref):
        s = jnp.sum((ids_ref[...] < HALF).astype(jnp.int32))
        o_ref[...] = jnp.full((1, 16), s, jnp.int32)
        z_ref[...] = jnp.zeros((ROWS_PT, D), jnp.float32)

    out, zeros2d = pl.pallas_call(
        body,
        out_shape=[
            jax.ShapeDtypeStruct((1, 16), jnp.int32),
            jax.ShapeDtypeStruct((ROWS_PT, D), jnp.float32),
        ],
    )(ids2d)
    return out.reshape(16), zeros2d


def _sc_segment_sum(E2d, ids3d, zeros2d, split16):
    mesh = plsc.VectorSubcoreMesh(core_axis_name="c", subcore_axis_name="s")

    @functools.partial(
        pl.kernel,
        out_type=jax.ShapeDtypeStruct((NUM_NODES, D), jnp.float32),
        mesh=mesh,
        scratch_types=[
            pltpu.VMEM((NBUF, CHUNK, D), jnp.float32),
            pltpu.VMEM((NBUF, 1, CHUNK), jnp.int32),
            pltpu.VMEM_SHARED((ACC_ROWS, D), jnp.float32),
            pltpu.VMEM((16,), jnp.int32),
            pltpu.SemaphoreType.DMA((NBUF,)),
            pltpu.SemaphoreType.DMA((NBUF,)),
        ],
    )
    def k(e_hbm, ids_hbm, zeros_hbm, split_hbm, out_hbm,
          rows_v, idx_v, acc_s, split_m, sem_r, sem_i):
        cid = lax.axis_index("c")
        sid = lax.axis_index("s")

        pltpu.sync_copy(split_hbm, split_m)
        split = split_m[...][0]
        lo_c = jnp.where(cid == 0, 0, split // CHUNK)
        hi_c = jnp.where(cid == 0, (split + CHUNK - 1) // CHUNK, NUM_CHUNKS)
        base = cid * HALF

        @pl.when(sid < NS - 1)
        def _():
            pltpu.sync_copy(
                zeros_hbm,
                acc_s.at[pl.ds(base + sid * ROWS_PT, ROWS_PT)],
            )

        @pl.when(sid == NS - 1)
        def _():
            pltpu.sync_copy(
                zeros_hbm.at[pl.ds(0, ROWS_PT_LAST)],
                acc_s.at[pl.ds(base + (NS - 1) * ROWS_PT, ROWS_PT_LAST)],
            )

        plsc.subcore_barrier()

        per = (hi_c - lo_c + NS - 1) // NS
        t_lo = lo_c + sid * per
        t_hi = jnp.minimum(t_lo + per, hi_c)

        def start(it, b):
            j = t_lo + it

            @pl.when(j < t_hi)
            def _():
                pltpu.async_copy(ids_hbm.at[j], idx_v.at[b], sem_i.at[b])
                pltpu.async_copy(
                    e_hbm.at[pl.ds(j * CHUNK, CHUNK)], rows_v.at[b], sem_r.at[b]
                )

        def finish(it, b):
            j = t_lo + it

            @pl.when(j < t_hi)
            def _():
                pltpu.make_async_copy(ids_hbm.at[j], idx_v.at[b], sem_i.at[b]).wait()
                pltpu.make_async_copy(
                    e_hbm.at[pl.ds(j * CHUNK, CHUNK)], rows_v.at[b], sem_r.at[b]
                ).wait()
                pltpu.sync_copy(rows_v.at[b], acc_s.at[idx_v.at[b, 0]], add=True)

        start(0, 0)
        start(1, 1)

        my_n = jnp.maximum(t_hi - t_lo, 0)

        def body(kk, _):
            for b in range(NBUF):
                it = kk * NBUF + b
                start(it + 2, (b + 2) % NBUF)
                finish(it, b)
            return ()

        lax.fori_loop(0, (my_n + NBUF - 1) // NBUF + 1, body, ())
        plsc.subcore_barrier()

        @pl.when(sid < NS - 1)
        def _():
            pltpu.sync_copy(
                acc_s.at[pl.ds(base + sid * ROWS_PT, ROWS_PT)],
                out_hbm.at[pl.ds(base + sid * ROWS_PT, ROWS_PT)],
            )

        @pl.when(sid == NS - 1)
        def _():
            pltpu.sync_copy(
                acc_s.at[pl.ds(base + (NS - 1) * ROWS_PT, ROWS_PT_LAST)],
                out_hbm.at[pl.ds(base + (NS - 1) * ROWS_PT, ROWS_PT_LAST)],
            )

    return k(E2d, ids3d, zeros2d, split16)


@jax.jit
def kernel(V_set, E_set, node_ids):
    E2d = E_set[0]
    ids3d = node_ids[0].reshape(NUM_CHUNKS, 1, CHUNK)
    split16, zeros2d = _split_and_zeros(node_ids[0].reshape(NUM_EDGES // D, D))
    out = _sc_segment_sum(E2d, ids3d, zeros2d, split16)
    return out[jnp.newaxis]
